# baseline (device time: 13242 ns/iter reference)
import jax
import jax.numpy as jnp
from jax import lax
from jax.experimental import pallas as pl
from jax.experimental.pallas import tpu as pltpu

N_DEV = 4
E_LOCAL = 4
E_TOT = 16
N_TOK = 512
D_IN = 256
D_OUT = 512
CAP = 25
CHUNK = N_TOK // N_DEV
SLOTS = 32
G = E_LOCAL * SLOTS


def kernel(x, router_W, route_idx, expert_W):
    del router_W

    def body(x_ref, idx_ref, ew_ref, out_ref,
             keep_ref, ranks_ref, ygall_ref, send_sems, recv_sems):
        p = lax.axis_index("i")

        barrier = pltpu.get_barrier_semaphore()
        for d in range(1, N_DEV):
            pl.semaphore_signal(
                barrier, inc=1,
                device_id=((p + d) % N_DEV,),
                device_id_type=pl.DeviceIdType.MESH,
            )
        pl.semaphore_wait(barrier, N_DEV - 1)

        idx = idx_ref[:, :]
        ecols = lax.broadcasted_iota(jnp.int32, (N_TOK, E_TOT), 1)
        ind = (idx == ecols).astype(jnp.float32)
        row = lax.broadcasted_iota(jnp.int32, (N_TOK, N_TOK), 0)
        col = lax.broadcasted_iota(jnp.int32, (N_TOK, N_TOK), 1)
        tri = (col < row).astype(jnp.bfloat16)
        ranks = jnp.dot(tri, ind.astype(jnp.bfloat16),
                        preferred_element_type=jnp.float32)
        keep = ind * (ranks < CAP).astype(jnp.float32)
        keep_ref[:, :] = keep
        ranks_ref[:, :] = ranks

        myrep = (lax.broadcasted_iota(jnp.int32, (E_TOT, G), 0)
                 == p * E_LOCAL
                 + lax.broadcasted_iota(jnp.int32, (E_TOT, G), 1) // SLOTS
                 ).astype(jnp.float32)
        keep_rep = jnp.dot(keep, myrep, preferred_element_type=jnp.float32)
        ranks_rep = jnp.dot(ranks, myrep, preferred_element_type=jnp.float32)
        rmod = (lax.broadcasted_iota(jnp.int32, (N_TOK, G), 1)
                % SLOTS).astype(jnp.float32)
        pm = (keep_rep * (ranks_rep == rmod).astype(jnp.float32)
              ).astype(jnp.bfloat16)

        xg = lax.dot_general(pm, x_ref[:, :].astype(jnp.bfloat16),
                             (((0,), (0,)), ((), ())),
                             preferred_element_type=jnp.float32)
        for j in range(E_LOCAL):
            ygall_ref[p, pl.ds(j * SLOTS, SLOTS), :] = jnp.dot(
                xg[j * SLOTS:(j + 1) * SLOTS, :], ew_ref[j],
                preferred_element_type=jnp.float32).astype(jnp.bfloat16)

        rdmas = []
        for d in (2, 1, 3):
            rdma = pltpu.make_async_remote_copy(
                src_ref=ygall_ref.at[p],
                dst_ref=ygall_ref.at[p],
                send_sem=send_sems.at[d - 1],
                recv_sem=recv_sems.at[d - 1],
                device_id=((p + d) % N_DEV,),
                device_id_type=pl.DeviceIdType.MESH,
            )
            rdma.start()
            rdmas.append(rdma)

        kc = keep_ref[pl.ds(p * CHUNK, CHUNK), :]
        rc = ranks_ref[pl.ds(p * CHUNK, CHUNK), :]
        allrep = (lax.broadcasted_iota(jnp.int32, (E_TOT, N_DEV * G), 0)
                  == lax.broadcasted_iota(jnp.int32, (E_TOT, N_DEV * G), 1)
                  // SLOTS).astype(jnp.float32)
        kcr = jnp.dot(kc, allrep, preferred_element_type=jnp.float32)
        rcr = jnp.dot(rc, allrep, preferred_element_type=jnp.float32)
        rmod2 = (lax.broadcasted_iota(jnp.int32, (CHUNK, N_DEV * G), 1)
                 % SLOTS).astype(jnp.float32)
        pc = (kcr * (rcr == rmod2).astype(jnp.float32)
              ).astype(jnp.bfloat16)

        for rdma in rdmas:
            rdma.wait_recv()
        ygflat = ygall_ref[:, :, :].reshape(N_DEV * G, D_OUT)
        out_ref[:, :] = jnp.dot(pc, ygflat,
                                preferred_element_type=jnp.float32)
        for rdma in rdmas:
            rdma.wait_send()

    return pl.pallas_call(
        body,
        out_shape=jax.ShapeDtypeStruct((CHUNK, D_OUT), jnp.float32),
        in_specs=[
            pl.BlockSpec(memory_space=pltpu.VMEM),
            pl.BlockSpec(memory_space=pltpu.VMEM),
            pl.BlockSpec(memory_space=pltpu.VMEM),
        ],
        out_specs=pl.BlockSpec(memory_space=pltpu.VMEM),
        scratch_shapes=[
            pltpu.VMEM((N_TOK, E_TOT), jnp.float32),
            pltpu.VMEM((N_TOK, E_TOT), jnp.float32),
            pltpu.VMEM((N_DEV, G, D_OUT), jnp.bfloat16),
            pltpu.SemaphoreType.DMA((N_DEV - 1,)),
            pltpu.SemaphoreType.DMA((N_DEV - 1,)),
        ],
        compiler_params=pltpu.CompilerParams(collective_id=0),
    )(x, route_idx, expert_W)


# device time: 12189 ns/iter; 1.0864x vs baseline; 1.0864x over previous
import jax
import jax.numpy as jnp
from jax import lax
from jax.experimental import pallas as pl
from jax.experimental.pallas import tpu as pltpu

N_DEV = 4
E_LOCAL = 4
E_TOT = 16
N_TOK = 512
D_IN = 256
D_OUT = 512
CAP = 25
CHUNK = N_TOK // N_DEV
SLOTS = 32
G = E_LOCAL * SLOTS


def kernel(x, router_W, route_idx, expert_W):
    del router_W

    def body(x_ref, idx_ref, ew_ref, out_ref,
             keep_ref, ranks_ref, ygall_ref, send_sems, recv_sems):
        p = lax.axis_index("i")

        barrier = pltpu.get_barrier_semaphore()
        for d in range(1, N_DEV):
            pl.semaphore_signal(
                barrier, inc=1,
                device_id=((p + d) % N_DEV,),
                device_id_type=pl.DeviceIdType.MESH,
            )

        idx = idx_ref[:, :]
        ecols = lax.broadcasted_iota(jnp.int32, (N_TOK, E_TOT), 1)
        ind = (idx == ecols).astype(jnp.float32)
        row = lax.broadcasted_iota(jnp.int32, (N_TOK, N_TOK), 0)
        col = lax.broadcasted_iota(jnp.int32, (N_TOK, N_TOK), 1)
        tri = (col < row).astype(jnp.bfloat16)
        ranks = jnp.dot(tri, ind.astype(jnp.bfloat16),
                        preferred_element_type=jnp.float32)
        keep = ind * (ranks < CAP).astype(jnp.float32)
        keep_ref[:, :] = keep
        ranks_ref[:, :] = ranks

        myrep = (lax.broadcasted_iota(jnp.int32, (E_TOT, G), 0)
                 == p * E_LOCAL
                 + lax.broadcasted_iota(jnp.int32, (E_TOT, G), 1) // SLOTS
                 ).astype(jnp.float32)
        keep_rep = jnp.dot(keep, myrep, preferred_element_type=jnp.float32)
        ranks_rep = jnp.dot(ranks, myrep, preferred_element_type=jnp.float32)
        rmod = (lax.broadcasted_iota(jnp.int32, (N_TOK, G), 1)
                % SLOTS).astype(jnp.float32)
        pm = (keep_rep * (ranks_rep == rmod).astype(jnp.float32)
              ).astype(jnp.bfloat16)

        xg = lax.dot_general(pm, x_ref[:, :].astype(jnp.bfloat16),
                             (((0,), (0,)), ((), ())),
                             preferred_element_type=jnp.float32)
        for j in range(E_LOCAL):
            ygall_ref[p, pl.ds(j * SLOTS, SLOTS), :] = jnp.dot(
                xg[j * SLOTS:(j + 1) * SLOTS, :], ew_ref[j],
                preferred_element_type=jnp.float32).astype(jnp.bfloat16)

        kc = keep_ref[pl.ds(p * CHUNK, CHUNK), :]
        rc = ranks_ref[pl.ds(p * CHUNK, CHUNK), :]
        allrep = (lax.broadcasted_iota(jnp.int32, (E_TOT, N_DEV * G), 0)
                  == lax.broadcasted_iota(jnp.int32, (E_TOT, N_DEV * G), 1)
                  // SLOTS).astype(jnp.float32)
        kcr = jnp.dot(kc, allrep, preferred_element_type=jnp.float32)
        rcr = jnp.dot(rc, allrep, preferred_element_type=jnp.float32)
        rmod2 = (lax.broadcasted_iota(jnp.int32, (CHUNK, N_DEV * G), 1)
                 % SLOTS).astype(jnp.float32)
        pc = (kcr * (rcr == rmod2).astype(jnp.float32)
              ).astype(jnp.bfloat16)

        pl.semaphore_wait(barrier, N_DEV - 1)
        rdmas = []
        for d in (2, 1, 3):
            rdma = pltpu.make_async_remote_copy(
                src_ref=ygall_ref.at[p],
                dst_ref=ygall_ref.at[p],
                send_sem=send_sems.at[d - 1],
                recv_sem=recv_sems.at[d - 1],
                device_id=((p + d) % N_DEV,),
                device_id_type=pl.DeviceIdType.MESH,
            )
            rdma.start()
            rdmas.append(rdma)

        for rdma in rdmas:
            rdma.wait_recv()
        ygflat = ygall_ref[:, :, :].reshape(N_DEV * G, D_OUT)
        out_ref[:, :] = jnp.dot(pc, ygflat,
                                preferred_element_type=jnp.float32)
        for rdma in rdmas:
            rdma.wait_send()

    return pl.pallas_call(
        body,
        out_shape=jax.ShapeDtypeStruct((CHUNK, D_OUT), jnp.float32),
        in_specs=[
            pl.BlockSpec(memory_space=pltpu.VMEM),
            pl.BlockSpec(memory_space=pltpu.VMEM),
            pl.BlockSpec(memory_space=pltpu.VMEM),
        ],
        out_specs=pl.BlockSpec(memory_space=pltpu.VMEM),
        scratch_shapes=[
            pltpu.VMEM((N_TOK, E_TOT), jnp.float32),
            pltpu.VMEM((N_TOK, E_TOT), jnp.float32),
            pltpu.VMEM((N_DEV, G, D_OUT), jnp.bfloat16),
            pltpu.SemaphoreType.DMA((N_DEV - 1,)),
            pltpu.SemaphoreType.DMA((N_DEV - 1,)),
        ],
        compiler_params=pltpu.CompilerParams(collective_id=0),
    )(x, route_idx, expert_W)


# device time: 11712 ns/iter; 1.1306x vs baseline; 1.0407x over previous
import jax
import jax.numpy as jnp
from jax import lax
from jax.experimental import pallas as pl
from jax.experimental.pallas import tpu as pltpu

N_DEV = 4
E_LOCAL = 4
E_TOT = 16
N_TOK = 512
D_IN = 256
D_OUT = 512
CAP = 25
CHUNK = N_TOK // N_DEV
SLOTS = CAP
G = E_LOCAL * SLOTS


def kernel(x, router_W, route_idx, expert_W):
    del router_W

    def body(x_ref, idx_ref, ew_ref, out_ref,
             keep_ref, ranks_ref, ygall_ref, send_sems, recv_sems):
        p = lax.axis_index("i")

        barrier = pltpu.get_barrier_semaphore()
        for d in range(1, N_DEV):
            pl.semaphore_signal(
                barrier, inc=1,
                device_id=((p + d) % N_DEV,),
                device_id_type=pl.DeviceIdType.MESH,
            )

        idx = idx_ref[:, :]
        ecols = lax.broadcasted_iota(jnp.int32, (N_TOK, E_TOT), 1)
        ind = (idx == ecols).astype(jnp.float32)
        row = lax.broadcasted_iota(jnp.int32, (N_TOK, N_TOK), 0)
        col = lax.broadcasted_iota(jnp.int32, (N_TOK, N_TOK), 1)
        tri = (col < row).astype(jnp.bfloat16)
        ranks = jnp.dot(tri, ind.astype(jnp.bfloat16),
                        preferred_element_type=jnp.float32)
        keep = ind * (ranks < CAP).astype(jnp.float32)
        keep_ref[:, :] = keep
        ranks_ref[:, :] = ranks

        myrep = (lax.broadcasted_iota(jnp.int32, (E_TOT, G), 0)
                 == p * E_LOCAL
                 + lax.broadcasted_iota(jnp.int32, (E_TOT, G), 1) // SLOTS
                 ).astype(jnp.float32)
        keep_rep = jnp.dot(keep, myrep, preferred_element_type=jnp.float32)
        ranks_rep = jnp.dot(ranks, myrep, preferred_element_type=jnp.float32)
        rmod = (lax.broadcasted_iota(jnp.int32, (N_TOK, G), 1)
                % SLOTS).astype(jnp.float32)
        pm = (keep_rep * (ranks_rep == rmod).astype(jnp.float32)
              ).astype(jnp.bfloat16)

        xg = lax.dot_general(pm, x_ref[:, :].astype(jnp.bfloat16),
                             (((0,), (0,)), ((), ())),
                             preferred_element_type=jnp.float32)
        for j in range(E_LOCAL):
            ygall_ref[p, pl.ds(j * SLOTS, SLOTS), :] = jnp.dot(
                xg[j * SLOTS:(j + 1) * SLOTS, :], ew_ref[j],
                preferred_element_type=jnp.float32).astype(jnp.bfloat16)

        kc = keep_ref[pl.ds(p * CHUNK, CHUNK), :]
        rc = ranks_ref[pl.ds(p * CHUNK, CHUNK), :]
        rmod2 = (lax.broadcasted_iota(jnp.int32, (CHUNK, G), 1)
                 % SLOTS).astype(jnp.float32)
        pcs = []
        for d in range(N_DEV):
            r = (p + d) % N_DEV
            rep = (lax.broadcasted_iota(jnp.int32, (E_TOT, G), 0)
                   == r * E_LOCAL
                   + lax.broadcasted_iota(jnp.int32, (E_TOT, G), 1) // SLOTS
                   ).astype(jnp.float32)
            kcr = jnp.dot(kc, rep, preferred_element_type=jnp.float32)
            rcr = jnp.dot(rc, rep, preferred_element_type=jnp.float32)
            pcs.append((kcr * (rcr == rmod2).astype(jnp.float32)
                        ).astype(jnp.bfloat16))

        pl.semaphore_wait(barrier, N_DEV - 1)
        rdma_by_d = {}
        for d in (2, 1, 3):
            rdma = pltpu.make_async_remote_copy(
                src_ref=ygall_ref.at[p],
                dst_ref=ygall_ref.at[p],
                send_sem=send_sems.at[d - 1],
                recv_sem=recv_sems.at[d - 1],
                device_id=((p + d) % N_DEV,),
                device_id_type=pl.DeviceIdType.MESH,
            )
            rdma.start()
            rdma_by_d[d] = rdma

        acc = jnp.dot(pcs[0], ygall_ref[p],
                      preferred_element_type=jnp.float32)
        for d in (1, 3, 2):
            rdma_by_d[d].wait_recv()
            acc = acc + jnp.dot(pcs[d], ygall_ref[(p + d) % N_DEV],
                                preferred_element_type=jnp.float32)
        out_ref[:, :] = acc
        for d in (2, 1, 3):
            rdma_by_d[d].wait_send()

    return pl.pallas_call(
        body,
        out_shape=jax.ShapeDtypeStruct((CHUNK, D_OUT), jnp.float32),
        in_specs=[
            pl.BlockSpec(memory_space=pltpu.VMEM),
            pl.BlockSpec(memory_space=pltpu.VMEM),
            pl.BlockSpec(memory_space=pltpu.VMEM),
        ],
        out_specs=pl.BlockSpec(memory_space=pltpu.VMEM),
        scratch_shapes=[
            pltpu.VMEM((N_TOK, E_TOT), jnp.float32),
            pltpu.VMEM((N_TOK, E_TOT), jnp.float32),
            pltpu.VMEM((N_DEV, G, D_OUT), jnp.bfloat16),
            pltpu.SemaphoreType.DMA((N_DEV - 1,)),
            pltpu.SemaphoreType.DMA((N_DEV - 1,)),
        ],
        compiler_params=pltpu.CompilerParams(collective_id=0),
    )(x, route_idx, expert_W)


# device time: 11697 ns/iter; 1.1321x vs baseline; 1.0013x over previous
import jax
import jax.numpy as jnp
from jax import lax
from jax.experimental import pallas as pl
from jax.experimental.pallas import tpu as pltpu

N_DEV = 4
E_LOCAL = 4
E_TOT = 16
N_TOK = 512
D_IN = 256
D_OUT = 512
CAP = 25
CHUNK = N_TOK // N_DEV
SLOTS = CAP
G = E_LOCAL * SLOTS


def kernel(x, router_W, route_idx, expert_W):
    del router_W

    def body(x_ref, idx_ref, ew_ref, out_ref,
             keep_ref, ranks_ref, ygall_ref, send_sems, recv_sems,
             ready_sems):
        p = lax.axis_index("i")

        for d in range(1, N_DEV):
            pl.semaphore_signal(
                ready_sems.at[p], inc=1,
                device_id=((p + d) % N_DEV,),
                device_id_type=pl.DeviceIdType.MESH,
            )
        barrier = pltpu.get_barrier_semaphore()
        pl.semaphore_signal(barrier, inc=1)
        pl.semaphore_wait(barrier, 1)

        idx = idx_ref[:, :]
        ecols = lax.broadcasted_iota(jnp.int32, (N_TOK, E_TOT), 1)
        ind = (idx == ecols).astype(jnp.float32)
        row = lax.broadcasted_iota(jnp.int32, (N_TOK, N_TOK), 0)
        col = lax.broadcasted_iota(jnp.int32, (N_TOK, N_TOK), 1)
        tri = (col < row).astype(jnp.bfloat16)
        ranks = jnp.dot(tri, ind.astype(jnp.bfloat16),
                        preferred_element_type=jnp.float32)
        keep = ind * (ranks < CAP).astype(jnp.float32)
        keep_ref[:, :] = keep
        ranks_ref[:, :] = ranks

        myrep = (lax.broadcasted_iota(jnp.int32, (E_TOT, G), 0)
                 == p * E_LOCAL
                 + lax.broadcasted_iota(jnp.int32, (E_TOT, G), 1) // SLOTS
                 ).astype(jnp.float32)
        keep_rep = jnp.dot(keep, myrep, preferred_element_type=jnp.float32)
        ranks_rep = jnp.dot(ranks, myrep, preferred_element_type=jnp.float32)
        rmod = (lax.broadcasted_iota(jnp.int32, (N_TOK, G), 1)
                % SLOTS).astype(jnp.float32)
        pm = (keep_rep * (ranks_rep == rmod).astype(jnp.float32)
              ).astype(jnp.bfloat16)

        xg = lax.dot_general(pm, x_ref[:, :].astype(jnp.bfloat16),
                             (((0,), (0,)), ((), ())),
                             preferred_element_type=jnp.float32)
        for j in range(E_LOCAL):
            ygall_ref[p, pl.ds(j * SLOTS, SLOTS), :] = jnp.dot(
                xg[j * SLOTS:(j + 1) * SLOTS, :], ew_ref[j],
                preferred_element_type=jnp.float32).astype(jnp.bfloat16)

        kc = keep_ref[pl.ds(p * CHUNK, CHUNK), :]
        rc = ranks_ref[pl.ds(p * CHUNK, CHUNK), :]
        rmod2 = (lax.broadcasted_iota(jnp.int32, (CHUNK, G), 1)
                 % SLOTS).astype(jnp.float32)
        pcs = []
        for d in range(N_DEV):
            r = (p + d) % N_DEV
            rep = (lax.broadcasted_iota(jnp.int32, (E_TOT, G), 0)
                   == r * E_LOCAL
                   + lax.broadcasted_iota(jnp.int32, (E_TOT, G), 1) // SLOTS
                   ).astype(jnp.float32)
            kcr = jnp.dot(kc, rep, preferred_element_type=jnp.float32)
            rcr = jnp.dot(rc, rep, preferred_element_type=jnp.float32)
            pcs.append((kcr * (rcr == rmod2).astype(jnp.float32)
                        ).astype(jnp.bfloat16))

        rdma_by_d = {}
        for d in (2, 1, 3):
            q = (p + d) % N_DEV
            rdma = pltpu.make_async_remote_copy(
                src_ref=ygall_ref.at[p],
                dst_ref=ygall_ref.at[p],
                send_sem=send_sems.at[d - 1],
                recv_sem=recv_sems.at[d - 1],
                device_id=(q,),
                device_id_type=pl.DeviceIdType.MESH,
            )
            pl.semaphore_wait(ready_sems.at[q], 1)
            rdma.start()
            rdma_by_d[d] = rdma

        acc = jnp.dot(pcs[0], ygall_ref[p],
                      preferred_element_type=jnp.float32)
        for d in (1, 3, 2):
            rdma_by_d[d].wait_recv()
            acc = acc + jnp.dot(pcs[d], ygall_ref[(p + d) % N_DEV],
                                preferred_element_type=jnp.float32)
        out_ref[:, :] = acc
        for d in (2, 1, 3):
            rdma_by_d[d].wait_send()

    return pl.pallas_call(
        body,
        out_shape=jax.ShapeDtypeStruct((CHUNK, D_OUT), jnp.float32),
        in_specs=[
            pl.BlockSpec(memory_space=pltpu.VMEM),
            pl.BlockSpec(memory_space=pltpu.VMEM),
            pl.BlockSpec(memory_space=pltpu.VMEM),
        ],
        out_specs=pl.BlockSpec(memory_space=pltpu.VMEM),
        scratch_shapes=[
            pltpu.VMEM((N_TOK, E_TOT), jnp.float32),
            pltpu.VMEM((N_TOK, E_TOT), jnp.float32),
            pltpu.VMEM((N_DEV, G, D_OUT), jnp.bfloat16),
            pltpu.SemaphoreType.DMA((N_DEV - 1,)),
            pltpu.SemaphoreType.DMA((N_DEV - 1,)),
            pltpu.SemaphoreType.REGULAR((N_DEV,)),
        ],
        compiler_params=pltpu.CompilerParams(collective_id=0),
    )(x, route_idx, expert_W)
